# baseline (speedup 1.0000x reference)
import jax
import jax.numpy as jnp
from jax import lax
from jax.experimental import pallas as pl
from jax.experimental.pallas import tpu as pltpu
from jax.experimental.pallas import tpu_sc as plsc

VOCAB = 1000000
EMB = 64
B = 4096
L = 200

NC = 2
NS = 16
NW = NC * NS

RPW = B // NW
RPC = 4
NCHUNK = RPW // RPC
NBUF = 2
SPLITS = ((0, 128), (128, 72))


def _gather_kernel(table, enc, out, idx_v, rows0, rows1, sg0, sg1, so0, so1):
    cid = lax.axis_index("c")
    sid = lax.axis_index("s")
    wid = sid * NC + cid
    row0 = wid * RPW

    rows = (rows0, rows1)
    sg = (sg0, sg1)
    so = (so0, so1)

    pltpu.sync_copy(enc.at[pl.ds(row0, RPW)], idx_v)

    def fire_gathers(c, b):
        for j in range(RPC):
            r = c * RPC + j
            for (off, n) in SPLITS:
                pltpu.async_copy(
                    table.at[idx_v.at[r, pl.ds(off, n)]],
                    rows[b].at[j, pl.ds(off, n)],
                    sg[b],
                )

    def wait_gathers(c, b):
        for j in range(RPC):
            r = c * RPC + j
            for (off, n) in SPLITS:
                pltpu.make_async_copy(
                    table.at[idx_v.at[r, pl.ds(off, n)]],
                    rows[b].at[j, pl.ds(off, n)],
                    sg[b],
                ).wait()

    def out_copy(c, b):
        return pltpu.make_async_copy(
            rows[b],
            out.at[pl.ds(row0 + c * RPC, RPC)],
            so[b],
        )

    for b in range(NBUF):
        fire_gathers(b, b)

    def body(i, carry):
        c0 = i * NBUF
        for b in range(NBUF):
            c = c0 + b
            wait_gathers(c, b)
            cp = out_copy(c, b)
            cp.start()
            cp.wait()

            @pl.when(c + NBUF < NCHUNK)
            def _():
                fire_gathers(c + NBUF, b)

        return carry

    lax.fori_loop(0, NCHUNK // NBUF, body, 0)


def _mask_body(enc_ref, mask_ref):
    mask_ref[...] = (enc_ref[...] != 0).astype(jnp.int32)


@jax.jit
def _run(encoded, embeddings):
    gather = pl.kernel(
        _gather_kernel,
        out_type=jax.ShapeDtypeStruct((B, L, EMB), jnp.float32),
        mesh=plsc.VectorSubcoreMesh(core_axis_name="c", subcore_axis_name="s"),
        compiler_params=pltpu.CompilerParams(use_tc_tiling_on_sc=False),
        scratch_types=[
            pltpu.VMEM((RPW, L), jnp.int32),
            pltpu.VMEM((RPC, L, EMB), jnp.float32),
            pltpu.VMEM((RPC, L, EMB), jnp.float32),
            pltpu.SemaphoreType.DMA,
            pltpu.SemaphoreType.DMA,
            pltpu.SemaphoreType.DMA,
            pltpu.SemaphoreType.DMA,
        ],
    )
    emb = gather(embeddings, encoded)
    mask = pl.pallas_call(
        _mask_body,
        out_shape=jax.ShapeDtypeStruct((B, L), jnp.int32),
    )(encoded)
    return emb, mask


def kernel(encoded, embeddings):
    return _run(encoded, embeddings)


# speedup vs baseline: 1.2175x; 1.2175x over previous
import jax
import jax.numpy as jnp
from jax import lax
from jax.experimental import pallas as pl
from jax.experimental.pallas import tpu as pltpu
from jax.experimental.pallas import tpu_sc as plsc

VOCAB = 1000000
EMB = 64
EMBP = 128
B = 4096
L = 200

NC = 2
NS = 16
NW = NC * NS

RPW = B // NW
RPC = 2
NCHUNK = RPW // RPC
NBUF = 2
SPLITS = ((0, 128), (128, 72))


def _gather_kernel(table, enc, out, idx_v, rows0, rows1, sg0, sg1, so0, so1):
    cid = lax.axis_index("c")
    sid = lax.axis_index("s")
    wid = sid * NC + cid
    row0 = wid * RPW

    rows = (rows0, rows1)
    sg = (sg0, sg1)
    so = (so0, so1)

    pltpu.sync_copy(enc.at[pl.ds(row0, RPW)], idx_v)

    def fire_gathers(c, b):
        for j in range(RPC):
            r = c * RPC + j
            for (off, n) in SPLITS:
                pltpu.async_copy(
                    table.at[idx_v.at[r, pl.ds(off, n)]],
                    rows[b].at[j, pl.ds(off, n)],
                    sg[b],
                )

    def wait_gathers(c, b):
        for j in range(RPC):
            r = c * RPC + j
            for (off, n) in SPLITS:
                pltpu.make_async_copy(
                    table.at[idx_v.at[r, pl.ds(off, n)]],
                    rows[b].at[j, pl.ds(off, n)],
                    sg[b],
                ).wait()

    def out_copy(c, b):
        return pltpu.make_async_copy(
            rows[b],
            out.at[pl.ds(row0 + c * RPC, RPC)],
            so[b],
        )

    for b in range(NBUF):
        fire_gathers(b, b)

    def body(i, carry):
        c0 = i * NBUF
        for b in range(NBUF):
            c = c0 + b
            wait_gathers(c, b)
            cp = out_copy(c, b)
            cp.start()
            cp.wait()

            @pl.when(c + NBUF < NCHUNK)
            def _():
                fire_gathers(c + NBUF, b)

        return carry

    lax.fori_loop(0, NCHUNK // NBUF, body, 0)


def _mask_body(enc_ref, mask_ref):
    mask_ref[...] = (enc_ref[...] != 0).astype(jnp.int32)


@jax.jit
def _run(encoded, embeddings):
    t128 = jnp.pad(embeddings, ((0, 0), (0, EMBP - EMB)))
    gather = pl.kernel(
        _gather_kernel,
        out_type=jax.ShapeDtypeStruct((B, L, EMBP), jnp.float32),
        mesh=plsc.VectorSubcoreMesh(core_axis_name="c", subcore_axis_name="s"),
        compiler_params=pltpu.CompilerParams(use_tc_tiling_on_sc=False),
        scratch_types=[
            pltpu.VMEM((RPW, L), jnp.int32),
            pltpu.VMEM((RPC, L, EMBP), jnp.float32),
            pltpu.VMEM((RPC, L, EMBP), jnp.float32),
            pltpu.SemaphoreType.DMA,
            pltpu.SemaphoreType.DMA,
            pltpu.SemaphoreType.DMA,
            pltpu.SemaphoreType.DMA,
        ],
    )
    out128 = gather(t128, encoded)
    emb = out128[:, :, :EMB]
    mask = pl.pallas_call(
        _mask_body,
        out_shape=jax.ShapeDtypeStruct((B, L), jnp.int32),
    )(encoded)
    return emb, mask


def kernel(encoded, embeddings):
    return _run(encoded, embeddings)
